# baseline (speedup 1.0000x reference)
import functools

import jax
import jax.numpy as jnp
from jax import lax
from jax.experimental import pallas as pl
from jax.experimental.pallas import tpu as pltpu
from jax.experimental.pallas import tpu_sc as plsc

N_FACTORS = 32
BATCH = 16384

_info = plsc.get_sparse_core_info()
_NC = _info.num_cores
_NS = _info.num_subcores
_L = _info.num_lanes
_NW = _NC * _NS
_BPW = BATCH // _NW
_CH = 128
_NCH = _BPW // _CH


def _mf_body(uidx_ref, iidx_ref, uemb_ref, iemb_ref, out_ref,
             uidx_v, iidx_v, urow_v, irow_v, out_v, sem):
    wid = lax.axis_index("s") * _NC + lax.axis_index("c")
    base = wid * _BPW
    crow = wid * _NCH

    pltpu.sync_copy(uidx_ref.at[pl.ds(crow, _NCH)], uidx_v)
    pltpu.sync_copy(iidx_ref.at[pl.ds(crow, _NCH)], iidx_v)

    cps = []
    for j in range(_NCH):
        cps.append(pltpu.async_copy(
            uemb_ref.at[uidx_v.at[j]], urow_v.at[pl.ds(j * _CH, _CH)], sem))
        cps.append(pltpu.async_copy(
            iemb_ref.at[iidx_v.at[j]], irow_v.at[pl.ds(j * _CH, _CH)], sem))
    for cp in cps:
        cp.wait()

    def group(g, carry):
        rows = g * _L + lax.iota(jnp.int32, _L)
        acc = jnp.zeros((_L,), jnp.float32)
        for d in range(N_FACTORS):
            cols = jnp.full((_L,), d, jnp.int32)
            u = plsc.load_gather(urow_v, [rows, cols])
            v = plsc.load_gather(irow_v, [rows, cols])
            acc = acc + u * v
        out_v[pl.ds(g * _L, _L)] = acc
        return carry

    lax.fori_loop(0, _BPW // _L, group, 0)

    pltpu.sync_copy(out_v, out_ref.at[pl.ds(base, _BPW)])


@functools.partial(
    pl.kernel,
    out_type=jax.ShapeDtypeStruct((BATCH,), jnp.float32),
    mesh=plsc.VectorSubcoreMesh(core_axis_name="c", subcore_axis_name="s"),
    compiler_params=pltpu.CompilerParams(
        needs_layout_passes=False, use_tc_tiling_on_sc=False),
    scratch_types=[
        pltpu.VMEM((_NCH, _CH), jnp.int32),
        pltpu.VMEM((_NCH, _CH), jnp.int32),
        pltpu.VMEM((_BPW, N_FACTORS), jnp.float32),
        pltpu.VMEM((_BPW, N_FACTORS), jnp.float32),
        pltpu.VMEM((_BPW,), jnp.float32),
        pltpu.SemaphoreType.DMA,
    ],
)
def _mf_kernel(uidx, iidx, uemb, iemb, out,
               uidx_v, iidx_v, urow_v, irow_v, out_v, sem):
    _mf_body(uidx, iidx, uemb, iemb, out,
             uidx_v, iidx_v, urow_v, irow_v, out_v, sem)


def kernel(user_idx, item_idx, user_emb, item_emb):
    uidx = user_idx.astype(jnp.int32).reshape(_NW * _NCH, _CH)
    iidx = item_idx.astype(jnp.int32).reshape(_NW * _NCH, _CH)
    return _mf_kernel(uidx, iidx, user_emb, item_emb)


# speedup vs baseline: 3.6248x; 3.6248x over previous
import functools

import jax
import jax.numpy as jnp
from jax import lax
from jax.experimental import pallas as pl
from jax.experimental.pallas import tpu as pltpu
from jax.experimental.pallas import tpu_sc as plsc

N_FACTORS = 32
BATCH = 16384

_info = plsc.get_sparse_core_info()
_NC = _info.num_cores
_NS = _info.num_subcores
_L = _info.num_lanes
_NW = _NC * _NS
_BPW = BATCH // _NW
_HG = 8


def _mf_body(uidx_ref, iidx_ref, uembT_ref, iembT_ref, out_ref,
             uidx_v, iidx_v, ubuf, vbuf, out_v, sem):
    wid = lax.axis_index("s") * _NC + lax.axis_index("c")
    base = wid * _BPW

    pltpu.sync_copy(uidx_ref.at[pl.ds(base, _BPW)], uidx_v)
    pltpu.sync_copy(iidx_ref.at[pl.ds(base, _BPW)], iidx_v)

    lane_iota = lax.iota(jnp.int32, _L)
    rows_lo = lane_iota
    rows_hi = lane_iota + _L

    def group(g, carry):
        vu = uidx_v[pl.ds(g * _L, _L)]
        vi = iidx_v[pl.ds(g * _L, _L)]
        acc = jnp.zeros((_L,), jnp.float32)
        for h in range(_L // _HG):
            cps = []
            for l in range(_HG):
                lane = h * _HG + l
                ru = vu[lane]
                ri = vi[lane]
                uoff = pl.multiple_of(ru - (ru & 127), 128)
                ioff = pl.multiple_of(ri - (ri & 127), 128)
                cps.append(pltpu.async_copy(
                    uembT_ref.at[:, pl.ds(uoff, 128)], ubuf.at[l], sem))
                cps.append(pltpu.async_copy(
                    iembT_ref.at[:, pl.ds(ioff, 128)], vbuf.at[l], sem))
            for cp in cps:
                cp.wait()
            for l in range(_HG):
                lane = h * _HG + l
                cu = jnp.full((_L,), vu[lane] & 127, jnp.int32)
                ci = jnp.full((_L,), vi[lane] & 127, jnp.int32)
                u0 = plsc.load_gather(ubuf.at[l], [rows_lo, cu])
                u1 = plsc.load_gather(ubuf.at[l], [rows_hi, cu])
                v0 = plsc.load_gather(vbuf.at[l], [rows_lo, ci])
                v1 = plsc.load_gather(vbuf.at[l], [rows_hi, ci])
                s = jnp.sum(u0 * v0 + u1 * v1)
                acc = jnp.where(lane_iota == lane, s, acc)
        out_v[pl.ds(g * _L, _L)] = acc
        return carry

    lax.fori_loop(0, _BPW // _L, group, 0)

    pltpu.sync_copy(out_v, out_ref.at[pl.ds(base, _BPW)])


@functools.partial(
    pl.kernel,
    out_type=jax.ShapeDtypeStruct((BATCH,), jnp.float32),
    mesh=plsc.VectorSubcoreMesh(core_axis_name="c", subcore_axis_name="s"),
    compiler_params=pltpu.CompilerParams(needs_layout_passes=False),
    scratch_types=[
        pltpu.VMEM((_BPW,), jnp.int32),
        pltpu.VMEM((_BPW,), jnp.int32),
        pltpu.VMEM((_HG, N_FACTORS, 128), jnp.float32),
        pltpu.VMEM((_HG, N_FACTORS, 128), jnp.float32),
        pltpu.VMEM((_BPW,), jnp.float32),
        pltpu.SemaphoreType.DMA,
    ],
)
def _mf_kernel(uidx, iidx, uembT, iembT, out,
               uidx_v, iidx_v, ubuf, vbuf, out_v, sem):
    _mf_body(uidx, iidx, uembT, iembT, out,
             uidx_v, iidx_v, ubuf, vbuf, out_v, sem)


def kernel(user_idx, item_idx, user_emb, item_emb):
    uidx = user_idx.astype(jnp.int32)
    iidx = item_idx.astype(jnp.int32)
    return _mf_kernel(uidx, iidx, user_emb.T, item_emb.T)


# speedup vs baseline: 3.6305x; 1.0016x over previous
import functools

import jax
import jax.numpy as jnp
from jax import lax
from jax.experimental import pallas as pl
from jax.experimental.pallas import tpu as pltpu
from jax.experimental.pallas import tpu_sc as plsc

N_FACTORS = 32
BATCH = 16384

_info = plsc.get_sparse_core_info()
_NC = _info.num_cores
_NS = _info.num_subcores
_L = _info.num_lanes
_NW = _NC * _NS
_BPW = BATCH // _NW
_HG = 8


def _mf_body(uidx_ref, iidx_ref, uembT_ref, iembT_ref, out_ref,
             uidx_v, iidx_v, ubuf, vbuf, out_v, sem):
    wid = lax.axis_index("s") * _NC + lax.axis_index("c")
    base = wid * _BPW

    pltpu.sync_copy(uidx_ref.at[pl.ds(base, _BPW)], uidx_v)
    pltpu.sync_copy(iidx_ref.at[pl.ds(base, _BPW)], iidx_v)

    lane_iota = lax.iota(jnp.int32, _L)
    rows_lo = lane_iota
    rows_hi = lane_iota + _L

    def group(g, carry):
        vu = uidx_v[pl.ds(g * _L, _L)]
        vi = iidx_v[pl.ds(g * _L, _L)]
        acc = jnp.zeros((_L,), jnp.float32)
        for h in range(_L // _HG):
            cps = []
            for l in range(_HG):
                lane = h * _HG + l
                ru = vu[lane]
                ri = vi[lane]
                uoff = pl.multiple_of(ru - (ru & 127), 128)
                ioff = pl.multiple_of(ri - (ri & 127), 128)
                for tr in range(N_FACTORS // 8):
                    rs = pl.ds(tr * 8, 8)
                    cps.append(pltpu.async_copy(
                        uembT_ref.at[rs, pl.ds(uoff, 128)],
                        ubuf.at[l, rs], sem))
                    cps.append(pltpu.async_copy(
                        iembT_ref.at[rs, pl.ds(ioff, 128)],
                        vbuf.at[l, rs], sem))
            for cp in cps:
                cp.wait()
            for l in range(_HG):
                lane = h * _HG + l
                cu = jnp.full((_L,), vu[lane] & 127, jnp.int32)
                ci = jnp.full((_L,), vi[lane] & 127, jnp.int32)
                u0 = plsc.load_gather(ubuf.at[l], [rows_lo, cu])
                u1 = plsc.load_gather(ubuf.at[l], [rows_hi, cu])
                v0 = plsc.load_gather(vbuf.at[l], [rows_lo, ci])
                v1 = plsc.load_gather(vbuf.at[l], [rows_hi, ci])
                s = jnp.sum(u0 * v0 + u1 * v1)
                acc = jnp.where(lane_iota == lane, s, acc)
        out_v[pl.ds(g * _L, _L)] = acc
        return carry

    lax.fori_loop(0, _BPW // _L, group, 0)

    pltpu.sync_copy(out_v, out_ref.at[pl.ds(base, _BPW)])


@functools.partial(
    pl.kernel,
    out_type=jax.ShapeDtypeStruct((BATCH,), jnp.float32),
    mesh=plsc.VectorSubcoreMesh(core_axis_name="c", subcore_axis_name="s"),
    compiler_params=pltpu.CompilerParams(needs_layout_passes=False),
    scratch_types=[
        pltpu.VMEM((_BPW,), jnp.int32),
        pltpu.VMEM((_BPW,), jnp.int32),
        pltpu.VMEM((_HG, N_FACTORS, 128), jnp.float32),
        pltpu.VMEM((_HG, N_FACTORS, 128), jnp.float32),
        pltpu.VMEM((_BPW,), jnp.float32),
        pltpu.SemaphoreType.DMA,
    ],
)
def _mf_kernel(uidx, iidx, uembT, iembT, out,
               uidx_v, iidx_v, ubuf, vbuf, out_v, sem):
    _mf_body(uidx, iidx, uembT, iembT, out,
             uidx_v, iidx_v, ubuf, vbuf, out_v, sem)


def kernel(user_idx, item_idx, user_emb, item_emb):
    uidx = user_idx.astype(jnp.int32)
    iidx = item_idx.astype(jnp.int32)
    return _mf_kernel(uidx, iidx, user_emb.T, item_emb.T)


# speedup vs baseline: 3.7022x; 1.0198x over previous
import functools

import jax
import jax.numpy as jnp
from jax import lax
from jax.experimental import pallas as pl
from jax.experimental.pallas import tpu as pltpu
from jax.experimental.pallas import tpu_sc as plsc

N_FACTORS = 32
BATCH = 16384

_info = plsc.get_sparse_core_info()
_NC = _info.num_cores
_NS = _info.num_subcores
_L = _info.num_lanes
_NW = _NC * _NS
_BPW = BATCH // _NW
_Q = 4
_NQ = _BPW // _Q


def _mf_body(uidx_ref, iidx_ref, uembT_ref, iembT_ref, out_ref,
             uidx_v, iidx_v, ubuf, vbuf, out_v, sem0, sem1):
    wid = lax.axis_index("s") * _NC + lax.axis_index("c")
    base = wid * _BPW

    pltpu.sync_copy(uidx_ref.at[pl.ds(base, _BPW)], uidx_v.at[pl.ds(0, _BPW)])
    pltpu.sync_copy(iidx_ref.at[pl.ds(base, _BPW)], iidx_v.at[pl.ds(0, _BPW)])

    lane_iota = lax.iota(jnp.int32, _L)
    rows_lo = lane_iota
    rows_hi = lane_iota + _L

    def fire(vu, vi, lo, slot, sem):
        cps = []
        for j in range(_Q):
            ru = vu[lo + j]
            ri = vi[lo + j]
            uoff = pl.multiple_of(ru - (ru & 127), 128)
            ioff = pl.multiple_of(ri - (ri & 127), 128)
            cps.append(pltpu.async_copy(
                uembT_ref.at[:, pl.ds(uoff, 128)], ubuf.at[slot, j], sem))
            cps.append(pltpu.async_copy(
                iembT_ref.at[:, pl.ds(ioff, 128)], vbuf.at[slot, j], sem))
        return cps

    def drain(cps):
        for cp in cps:
            cp.wait()

    def compute(vu, vi, lo, slot, tc, acc):
        lbase = 4 * (tc & 3)
        for j in range(_Q):
            cu = jnp.full((_L,), vu[lo + j] & 127, jnp.int32)
            ci = jnp.full((_L,), vi[lo + j] & 127, jnp.int32)
            u0 = plsc.load_gather(ubuf.at[slot, j], [rows_lo, cu])
            u1 = plsc.load_gather(ubuf.at[slot, j], [rows_hi, cu])
            v0 = plsc.load_gather(vbuf.at[slot, j], [rows_lo, ci])
            v1 = plsc.load_gather(vbuf.at[slot, j], [rows_hi, ci])
            s = jnp.sum(u0 * v0 + u1 * v1)
            acc = jnp.where(lane_iota == lbase + j, s, acc)
        return acc

    def flush(tc, acc):
        @pl.when((tc & 3) == 3)
        def _():
            out_v[pl.ds((tc >> 2) * _L, _L)] = acc
        return jnp.where((tc & 3) == 3, jnp.zeros((_L,), jnp.float32), acc)

    def body(q, carry):
        vu_prev, vi_prev, acc = carry
        vu = uidx_v[pl.ds(q * 8, _L)]
        vi = iidx_v[pl.ds(q * 8, _L)]
        cps_a = fire(vu, vi, 0, 0, sem0)

        def prev_work(acc):
            for _ in range(8):
                pltpu.make_async_copy(
                    uembT_ref.at[:, pl.ds(0, 128)], ubuf.at[1, 0], sem1).wait()
            acc = compute(vu_prev, vi_prev, 4, 1, 2 * q - 1, acc)
            return flush(2 * q - 1, acc)

        acc = lax.cond(q > 0, prev_work, lambda a: a, acc)
        cps_b = fire(vu, vi, 4, 1, sem1)
        drain(cps_a)
        acc = compute(vu, vi, 0, 0, 2 * q, acc)
        acc = flush(2 * q, acc)
        del cps_b
        return vu, vi, acc

    vu_l, vi_l, acc = lax.fori_loop(
        0, _NQ // 2, body,
        (jnp.zeros((_L,), jnp.int32), jnp.zeros((_L,), jnp.int32),
         jnp.zeros((_L,), jnp.float32)))

    for _ in range(8):
        pltpu.make_async_copy(
            uembT_ref.at[:, pl.ds(0, 128)], ubuf.at[1, 0], sem1).wait()
    acc = compute(vu_l, vi_l, 4, 1, _NQ - 1, acc)
    out_v[pl.ds((_NQ // 4 - 1) * _L, _L)] = acc

    pltpu.sync_copy(out_v, out_ref.at[pl.ds(base, _BPW)])


@functools.partial(
    pl.kernel,
    out_type=jax.ShapeDtypeStruct((BATCH,), jnp.float32),
    mesh=plsc.VectorSubcoreMesh(core_axis_name="c", subcore_axis_name="s"),
    compiler_params=pltpu.CompilerParams(needs_layout_passes=False),
    scratch_types=[
        pltpu.VMEM((_BPW + _L,), jnp.int32),
        pltpu.VMEM((_BPW + _L,), jnp.int32),
        pltpu.VMEM((2, _Q, N_FACTORS, 128), jnp.float32),
        pltpu.VMEM((2, _Q, N_FACTORS, 128), jnp.float32),
        pltpu.VMEM((_BPW,), jnp.float32),
        pltpu.SemaphoreType.DMA,
        pltpu.SemaphoreType.DMA,
    ],
)
def _mf_kernel(uidx, iidx, uembT, iembT, out,
               uidx_v, iidx_v, ubuf, vbuf, out_v, sem0, sem1):
    _mf_body(uidx, iidx, uembT, iembT, out,
             uidx_v, iidx_v, ubuf, vbuf, out_v, sem0, sem1)


def kernel(user_idx, item_idx, user_emb, item_emb):
    uidx = user_idx.astype(jnp.int32)
    iidx = item_idx.astype(jnp.int32)
    return _mf_kernel(uidx, iidx, user_emb.T, item_emb.T)
